# Initial kernel scaffold; baseline (speedup 1.0000x reference)
import jax
import jax.numpy as jnp
from jax.experimental import pallas as pl


def kernel(x, edge_index, H, W1, b1, W2, b2):
    raise NotImplementedError("write your pallas kernel here")


# speedup vs baseline: 23.7114x; 23.7114x over previous
import functools

import jax
import jax.numpy as jnp
from jax import lax
from jax.experimental import pallas as pl
from jax.experimental.pallas import tpu as pltpu
from jax.experimental.pallas import tpu_sc as plsc

N = 10000
D = 128
E = 320000
NC = 2
NS = 16
CHUNK = 128
NBUF = 2

NPAD = 10240
DEG_PAD = 10240
E_PAD = 327680
ECHUNKS = E_PAD // CHUNK
CHUNKS_C = E_PAD // (NS * CHUNK)
CHUNKS_E = E_PAD // (NC * NS * CHUNK)
ZROWS = NPAD // NS
OROWS = NPAD // NS
DEG_T = DEG_PAD // NS

_mesh = lambda: plsc.VectorSubcoreMesh(
    core_axis_name="c", subcore_axis_name="s", num_cores=NC, num_subcores=NS)


def _deg_body(dst_hbm, ones_hbm, zeros_hbm, out_hbm, dst_idx, ones_v, acc):
  cid = lax.axis_index("c")
  tid = lax.axis_index("s")
  pltpu.sync_copy(zeros_hbm, acc.at[pl.ds(tid * DEG_T, DEG_T)])
  pltpu.sync_copy(ones_hbm, ones_v)
  row0 = (cid * NS + tid) * CHUNKS_E
  pltpu.sync_copy(dst_hbm.at[pl.ds(row0, CHUNKS_E)], dst_idx)
  plsc.subcore_barrier()

  @pl.loop(0, CHUNKS_E)
  def _(j):
    pltpu.sync_copy(ones_v, acc.at[dst_idx.at[j]], add=True)

  plsc.subcore_barrier()
  pltpu.sync_copy(acc.at[pl.ds(tid * DEG_T, DEG_T)],
                  out_hbm.at[pl.ds((cid * NS + tid) * DEG_T, DEG_T)])


def _make_deg_kernel():
  return pl.kernel(
      _deg_body,
      out_type=jax.ShapeDtypeStruct((NC * DEG_PAD,), jnp.float32),
      mesh=_mesh(),
      scratch_types=[
          pltpu.VMEM((CHUNKS_E, CHUNK), jnp.int32),
          pltpu.VMEM((CHUNK,), jnp.float32),
          pltpu.VMEM_SHARED((DEG_PAD,), jnp.float32),
      ],
  )


G = 16


def _prop_body(table_hbm, src_hbm, dst_hbm, zrows_hbm, out_hbm,
               src_idx, dst_idx, rows, acc, sem, *,
               chunks, src_core_rows, dst_core_rows):
  cid = lax.axis_index("c")
  tid = lax.axis_index("s")
  pltpu.sync_copy(zrows_hbm, acc.at[pl.ds(tid * ZROWS, ZROWS)])
  src_row0 = cid * src_core_rows + tid * chunks
  dst_row0 = cid * dst_core_rows + tid * chunks
  plsc.subcore_barrier()

  @pl.loop(0, chunks // G)
  def _(s):
    pltpu.sync_copy(src_hbm.at[pl.ds(src_row0 + s * G, G)], src_idx)
    pltpu.sync_copy(dst_hbm.at[pl.ds(dst_row0 + s * G, G)], dst_idx)
    for b in range(NBUF):
      pltpu.async_copy(table_hbm.at[src_idx.at[b]], rows.at[b], sem)

    @pl.loop(0, G // NBUF)
    def _(k):
      for b in range(NBUF):
        j = k * NBUF + b
        pltpu.make_async_copy(
            table_hbm.at[src_idx.at[j]], rows.at[b], sem).wait()
        pltpu.sync_copy(rows.at[b], acc.at[dst_idx.at[j]], add=True)
        nj = j + NBUF

        @pl.when(nj < G)
        def _():
          pltpu.async_copy(table_hbm.at[src_idx.at[nj]], rows.at[b], sem)

  plsc.subcore_barrier()
  pltpu.sync_copy(acc.at[pl.ds(tid * OROWS, OROWS)],
                  out_hbm.at[cid, pl.ds(tid * OROWS, OROWS)])


def _make_prop_kernel(chunks, src_core_rows, dst_core_rows):
  body = functools.partial(
      _prop_body, chunks=chunks, src_core_rows=src_core_rows,
      dst_core_rows=dst_core_rows)
  return pl.kernel(
      body,
      out_type=jax.ShapeDtypeStruct((NC, NPAD, D), jnp.float32),
      mesh=_mesh(),
      scratch_types=[
          pltpu.VMEM((G, CHUNK), jnp.int32),
          pltpu.VMEM((G, CHUNK), jnp.int32),
          pltpu.VMEM((NBUF, CHUNK, D), jnp.float32),
          pltpu.VMEM_SHARED((NPAD, D), jnp.float32),
          pltpu.SemaphoreType.DMA,
      ],
  )


BM = 1000


def _tc_scale_body(x_ref, h_ref, deg_ref, out_ref):
  dis = lax.rsqrt(deg_ref[...])
  out_ref[0] = x_ref[...] * dis
  out_ref[1] = h_ref[...] * dis


def _tc_scale(x, h, deg_col):
  return pl.pallas_call(
      _tc_scale_body,
      grid=(N // BM,),
      in_specs=[
          pl.BlockSpec((BM, D), lambda i: (i, 0)),
          pl.BlockSpec((BM, D), lambda i: (i, 0)),
          pl.BlockSpec((BM, 1), lambda i: (i, 0)),
      ],
      out_specs=pl.BlockSpec((NC, BM, D), lambda i: (0, i, 0)),
      out_shape=jax.ShapeDtypeStruct((NC, N, D), jnp.float32),
  )(x, h, deg_col)


def _tc_gates_body(acc0_ref, acc1_ref, v20_ref, v21_ref, h_ref, deg_ref,
                   w1_ref, b1_ref, w2_ref, b2_ref, t1_ref, u_ref, hrs_ref):
  dis = lax.rsqrt(deg_ref[...])
  px = (acc0_ref[0] + v20_ref[0]) * dis
  ph = (acc1_ref[0] + v21_ref[0]) * dis
  w1 = w1_ref[...]
  ru = jax.nn.sigmoid(
      jnp.dot(px, w1[:D], preferred_element_type=jnp.float32)
      + jnp.dot(ph, w1[D:], preferred_element_type=jnp.float32)
      + b1_ref[...])
  r = ru[:, :D]
  u = ru[:, D:]
  t1_ref[...] = (jnp.dot(px, w2_ref[...][:D], preferred_element_type=jnp.float32)
                 + b2_ref[...])
  u_ref[...] = u
  hrs_ref[...] = h_ref[...] * r * dis


def _tc_gates(accC, v2, h, deg_col, w1, b1, w2, b2):
  return pl.pallas_call(
      _tc_gates_body,
      grid=(N // BM,),
      in_specs=[
          pl.BlockSpec((1, BM, D), lambda i: (0, i, 0)),
          pl.BlockSpec((1, BM, D), lambda i: (1, i, 0)),
          pl.BlockSpec((1, BM, D), lambda i: (0, i, 0)),
          pl.BlockSpec((1, BM, D), lambda i: (1, i, 0)),
          pl.BlockSpec((BM, D), lambda i: (i, 0)),
          pl.BlockSpec((BM, 1), lambda i: (i, 0)),
          pl.BlockSpec((2 * D, 2 * D), lambda i: (0, 0)),
          pl.BlockSpec((2 * D,), lambda i: (0,)),
          pl.BlockSpec((2 * D, D), lambda i: (0, 0)),
          pl.BlockSpec((D,), lambda i: (0,)),
      ],
      out_specs=[
          pl.BlockSpec((BM, D), lambda i: (i, 0)),
          pl.BlockSpec((BM, D), lambda i: (i, 0)),
          pl.BlockSpec((BM, D), lambda i: (i, 0)),
      ],
      out_shape=[
          jax.ShapeDtypeStruct((N, D), jnp.float32),
          jax.ShapeDtypeStruct((N, D), jnp.float32),
          jax.ShapeDtypeStruct((N, D), jnp.float32),
      ],
  )(accC, accC, v2, v2, h, deg_col, w1, b1, w2, b2)


def _tc_out_body(accE0_ref, accE1_ref, hrs_ref, t1_ref, u_ref, h_ref, deg_ref,
                 w2_ref, out_ref):
  dis = lax.rsqrt(deg_ref[...])
  phr = (accE0_ref[0] + accE1_ref[0] + hrs_ref[...]) * dis
  c = jnp.tanh(t1_ref[...]
               + jnp.dot(phr, w2_ref[...][D:], preferred_element_type=jnp.float32))
  u = u_ref[...]
  out_ref[...] = u * h_ref[...] + (1.0 - u) * c


def _tc_out(accE, hrs, t1, u, h, deg_col, w2):
  return pl.pallas_call(
      _tc_out_body,
      grid=(N // BM,),
      in_specs=[
          pl.BlockSpec((1, BM, D), lambda i: (0, i, 0)),
          pl.BlockSpec((1, BM, D), lambda i: (1, i, 0)),
          pl.BlockSpec((BM, D), lambda i: (i, 0)),
          pl.BlockSpec((BM, D), lambda i: (i, 0)),
          pl.BlockSpec((BM, D), lambda i: (i, 0)),
          pl.BlockSpec((BM, D), lambda i: (i, 0)),
          pl.BlockSpec((BM, 1), lambda i: (i, 0)),
          pl.BlockSpec((2 * D, D), lambda i: (0, 0)),
      ],
      out_specs=pl.BlockSpec((BM, D), lambda i: (i, 0)),
      out_shape=jax.ShapeDtypeStruct((N, D), jnp.float32),
  )(accE, accE, hrs, t1, u, h, deg_col, w2)


def kernel(x, edge_index, H, W1, b1, W2, b2):
  src = edge_index[0].astype(jnp.int32)
  dst = edge_index[1].astype(jnp.int32)
  npe = E_PAD - E
  pad_i = jnp.arange(npe, dtype=jnp.int32)
  src_p = jnp.concatenate([src, (pad_i * 37) % N])
  dst_p = jnp.concatenate([dst, N + (pad_i % (NPAD - N))])
  src2 = src_p.reshape(ECHUNKS, CHUNK)
  srcC2 = jnp.concatenate([src_p, src_p + N]).reshape(2 * ECHUNKS, CHUNK)
  dst2 = dst_p.reshape(ECHUNKS, CHUNK)

  ones_v = jnp.ones((CHUNK,), jnp.float32)
  zeros_deg = jnp.zeros((DEG_T,), jnp.float32)
  zrows = jnp.zeros((ZROWS, D), jnp.float32)

  deg_parts = _make_deg_kernel()(dst2, ones_v, zeros_deg)
  deg = deg_parts.reshape(NC, DEG_PAD)[:, :N].sum(axis=0) + 1.0
  deg_col = deg[:, None]

  v2 = _tc_scale(x, H, deg_col)
  accC = _make_prop_kernel(CHUNKS_C, ECHUNKS, 0)(
      v2.reshape(NC * N, D), srcC2, dst2, zrows)
  t1, u, hrs = _tc_gates(accC, v2, H, deg_col, W1, b1, W2, b2)
  accE = _make_prop_kernel(CHUNKS_E, ECHUNKS // NC, ECHUNKS // NC)(
      hrs, src2, dst2, zrows)
  return _tc_out(accE, hrs, t1, u, H, deg_col, W2)


# speedup vs baseline: 26.7862x; 1.1297x over previous
import functools

import jax
import jax.numpy as jnp
from jax import lax
from jax.experimental import pallas as pl
from jax.experimental.pallas import tpu as pltpu
from jax.experimental.pallas import tpu_sc as plsc

N = 10000
D = 128
E = 320000
NC = 2
NS = 16
CHUNK = 128
NBUF = 2

NPAD = 10240
DEG_PAD = 10240
E_PAD = 327680
ECHUNKS = E_PAD // CHUNK
CHUNKS_C = E_PAD // (NS * CHUNK)
CHUNKS_E = E_PAD // (NC * NS * CHUNK)
ZROWS = NPAD // NS
OROWS = NPAD // NS
DEG_T = DEG_PAD // NS

_mesh = lambda: plsc.VectorSubcoreMesh(
    core_axis_name="c", subcore_axis_name="s", num_cores=NC, num_subcores=NS)


def _deg_body(ei_hbm, ones_hbm, zeros_hbm, out_hbm, dst_idx, ones_v, acc, sem):
  cid = lax.axis_index("c")
  tid = lax.axis_index("s")
  pltpu.sync_copy(zeros_hbm, acc.at[pl.ds(tid * DEG_T, DEG_T)])
  pltpu.sync_copy(ones_hbm, ones_v)
  row0 = (cid * NS + tid) * CHUNKS_E
  pltpu.sync_copy(ei_hbm.at[1, pl.ds(row0, CHUNKS_E)], dst_idx)
  plsc.subcore_barrier()

  @pl.loop(0, CHUNKS_E // 8)
  def _(g):
    for b in range(8):
      pltpu.async_copy(ones_v, acc.at[dst_idx.at[g * 8 + b]], sem, add=True)
    for b in range(8):
      pltpu.make_async_copy(ones_v, acc.at[dst_idx.at[g * 8 + b]], sem).wait()

  plsc.subcore_barrier()
  pltpu.sync_copy(acc.at[pl.ds(tid * DEG_T, DEG_T)],
                  out_hbm.at[pl.ds((cid * NS + tid) * DEG_T, DEG_T)])


def _make_deg_kernel():
  return pl.kernel(
      _deg_body,
      out_type=jax.ShapeDtypeStruct((NC * DEG_PAD,), jnp.float32),
      mesh=_mesh(),
      scratch_types=[
          pltpu.VMEM((CHUNKS_E, CHUNK), jnp.int32),
          pltpu.VMEM((CHUNK,), jnp.float32),
          pltpu.VMEM_SHARED((DEG_PAD,), jnp.float32),
          pltpu.SemaphoreType.DMA,
      ],
  )


G = 16


def _prop_body(table_hbm, ei_hbm, zrows_hbm, out_hbm,
               src_idx, dst_idx, rows, acc, sem, sem_s, *,
               chunks, core_rows, table_cores):
  cid = lax.axis_index("c")
  tid = lax.axis_index("s")
  pltpu.sync_copy(zrows_hbm, acc.at[pl.ds(tid * ZROWS, ZROWS)])
  row0 = cid * core_rows + tid * chunks
  plsc.subcore_barrier()

  nsc = chunks // G
  tbl = table_hbm.at[cid] if table_cores else table_hbm
  pltpu.sync_copy(ei_hbm.at[0, pl.ds(row0, G)], src_idx.at[0])
  pltpu.sync_copy(ei_hbm.at[1, pl.ds(row0, G)], dst_idx.at[0])
  for b in range(NBUF):
    pltpu.async_copy(tbl.at[src_idx.at[0, b]], rows.at[b], sem)

  @pl.loop(0, nsc)
  def _(s):
    p = s % 2

    @pl.when(s + 1 < nsc)
    def _():
      pltpu.async_copy(ei_hbm.at[0, pl.ds(row0 + (s + 1) * G, G)],
                       src_idx.at[1 - p], sem_s)
      pltpu.async_copy(ei_hbm.at[1, pl.ds(row0 + (s + 1) * G, G)],
                       dst_idx.at[1 - p], sem_s)

    @pl.loop(0, (G - NBUF) // NBUF)
    def _(k):
      for b in range(NBUF):
        j = k * NBUF + b
        pltpu.make_async_copy(
            tbl.at[src_idx.at[p, j]], rows.at[b], sem).wait()
        pltpu.sync_copy(rows.at[b], acc.at[dst_idx.at[p, j]], add=True)
        pltpu.async_copy(tbl.at[src_idx.at[p, j + NBUF]], rows.at[b], sem)

    @pl.when(s + 1 < nsc)
    def _():
      pltpu.make_async_copy(ei_hbm.at[0, pl.ds(row0 + (s + 1) * G, G)],
                            src_idx.at[1 - p], sem_s).wait()
      pltpu.make_async_copy(ei_hbm.at[1, pl.ds(row0 + (s + 1) * G, G)],
                            dst_idx.at[1 - p], sem_s).wait()

    for b in range(NBUF):
      j = G - NBUF + b
      pltpu.make_async_copy(
          tbl.at[src_idx.at[p, j]], rows.at[b], sem).wait()
      pltpu.sync_copy(rows.at[b], acc.at[dst_idx.at[p, j]], add=True)

      @pl.when(s + 1 < nsc)
      def _():
        pltpu.async_copy(tbl.at[src_idx.at[1 - p, b]], rows.at[b], sem)

  plsc.subcore_barrier()
  pltpu.sync_copy(acc.at[pl.ds(tid * OROWS, OROWS)],
                  out_hbm.at[cid, pl.ds(tid * OROWS, OROWS)])


def _make_prop_kernel(chunks, core_rows, table_cores):
  body = functools.partial(
      _prop_body, chunks=chunks, core_rows=core_rows, table_cores=table_cores)
  return pl.kernel(
      body,
      out_type=jax.ShapeDtypeStruct((NC, NPAD, D), jnp.float32),
      mesh=_mesh(),
      scratch_types=[
          pltpu.VMEM((2, G, CHUNK), jnp.int32),
          pltpu.VMEM((2, G, CHUNK), jnp.int32),
          pltpu.VMEM((NBUF, CHUNK, D), jnp.float32),
          pltpu.VMEM_SHARED((NPAD, D), jnp.float32),
          pltpu.SemaphoreType.DMA,
          pltpu.SemaphoreType.DMA,
      ],
  )


BM = 2000


def _tc_scale_body(x_ref, h_ref, deg_ref, out_ref):
  dis = lax.rsqrt(deg_ref[...])
  out_ref[0] = x_ref[...] * dis
  out_ref[1] = h_ref[...] * dis


def _tc_scale(x, h, deg_col):
  return pl.pallas_call(
      _tc_scale_body,
      grid=(N // BM,),
      in_specs=[
          pl.BlockSpec((BM, D), lambda i: (i, 0)),
          pl.BlockSpec((BM, D), lambda i: (i, 0)),
          pl.BlockSpec((BM, 1), lambda i: (i, 0)),
      ],
      out_specs=pl.BlockSpec((NC, BM, D), lambda i: (0, i, 0)),
      out_shape=jax.ShapeDtypeStruct((NC, N, D), jnp.float32),
  )(x, h, deg_col)


def _tc_gates_body(acc0_ref, acc1_ref, v20_ref, v21_ref, h_ref, deg_ref,
                   w1_ref, b1_ref, w2_ref, b2_ref, t1_ref, u_ref, hrs_ref):
  dis = lax.rsqrt(deg_ref[...])
  px = (acc0_ref[0] + v20_ref[0]) * dis
  ph = (acc1_ref[0] + v21_ref[0]) * dis
  w1 = w1_ref[...]
  ru = jax.nn.sigmoid(
      jnp.dot(px, w1[:D], preferred_element_type=jnp.float32)
      + jnp.dot(ph, w1[D:], preferred_element_type=jnp.float32)
      + b1_ref[...])
  r = ru[:, :D]
  u = ru[:, D:]
  t1_ref[...] = (jnp.dot(px, w2_ref[...][:D], preferred_element_type=jnp.float32)
                 + b2_ref[...])
  u_ref[...] = u
  hrs_ref[...] = h_ref[...] * r * dis


def _tc_gates(accC, v2, h, deg_col, w1, b1, w2, b2):
  return pl.pallas_call(
      _tc_gates_body,
      grid=(N // BM,),
      in_specs=[
          pl.BlockSpec((1, BM, D), lambda i: (0, i, 0)),
          pl.BlockSpec((1, BM, D), lambda i: (1, i, 0)),
          pl.BlockSpec((1, BM, D), lambda i: (0, i, 0)),
          pl.BlockSpec((1, BM, D), lambda i: (1, i, 0)),
          pl.BlockSpec((BM, D), lambda i: (i, 0)),
          pl.BlockSpec((BM, 1), lambda i: (i, 0)),
          pl.BlockSpec((2 * D, 2 * D), lambda i: (0, 0)),
          pl.BlockSpec((2 * D,), lambda i: (0,)),
          pl.BlockSpec((2 * D, D), lambda i: (0, 0)),
          pl.BlockSpec((D,), lambda i: (0,)),
      ],
      out_specs=[
          pl.BlockSpec((BM, D), lambda i: (i, 0)),
          pl.BlockSpec((BM, D), lambda i: (i, 0)),
          pl.BlockSpec((BM, D), lambda i: (i, 0)),
      ],
      out_shape=[
          jax.ShapeDtypeStruct((N, D), jnp.float32),
          jax.ShapeDtypeStruct((N, D), jnp.float32),
          jax.ShapeDtypeStruct((N, D), jnp.float32),
      ],
  )(accC, accC, v2, v2, h, deg_col, w1, b1, w2, b2)


def _tc_out_body(accE0_ref, accE1_ref, hrs_ref, t1_ref, u_ref, h_ref, deg_ref,
                 w2_ref, out_ref):
  dis = lax.rsqrt(deg_ref[...])
  phr = (accE0_ref[0] + accE1_ref[0] + hrs_ref[...]) * dis
  c = jnp.tanh(t1_ref[...]
               + jnp.dot(phr, w2_ref[...][D:], preferred_element_type=jnp.float32))
  u = u_ref[...]
  out_ref[...] = u * h_ref[...] + (1.0 - u) * c


def _tc_out(accE, hrs, t1, u, h, deg_col, w2):
  return pl.pallas_call(
      _tc_out_body,
      grid=(N // BM,),
      in_specs=[
          pl.BlockSpec((1, BM, D), lambda i: (0, i, 0)),
          pl.BlockSpec((1, BM, D), lambda i: (1, i, 0)),
          pl.BlockSpec((BM, D), lambda i: (i, 0)),
          pl.BlockSpec((BM, D), lambda i: (i, 0)),
          pl.BlockSpec((BM, D), lambda i: (i, 0)),
          pl.BlockSpec((BM, D), lambda i: (i, 0)),
          pl.BlockSpec((BM, 1), lambda i: (i, 0)),
          pl.BlockSpec((2 * D, D), lambda i: (0, 0)),
      ],
      out_specs=pl.BlockSpec((BM, D), lambda i: (i, 0)),
      out_shape=jax.ShapeDtypeStruct((N, D), jnp.float32),
  )(accE, accE, hrs, t1, u, h, deg_col, w2)


def kernel(x, edge_index, H, W1, b1, W2, b2):
  npe = E_PAD - E
  pad_i = jnp.arange(npe, dtype=jnp.int32)
  pad = jnp.stack([(pad_i * 37) % N, N + (pad_i % (NPAD - N))])
  ei2 = jnp.concatenate(
      [edge_index.astype(jnp.int32).reshape(2, E // CHUNK, CHUNK),
       pad.reshape(2, npe // CHUNK, CHUNK)], axis=1)

  ones_v = jnp.ones((CHUNK,), jnp.float32)
  zeros_deg = jnp.zeros((DEG_T,), jnp.float32)
  zrows = jnp.zeros((ZROWS, D), jnp.float32)

  deg_parts = _make_deg_kernel()(ei2, ones_v, zeros_deg)
  deg = deg_parts.reshape(NC, DEG_PAD)[:, :N].sum(axis=0) + 1.0
  deg_col = deg[:, None]

  v2 = _tc_scale(x, H, deg_col)
  accC = _make_prop_kernel(CHUNKS_C, 0, True)(
      v2, ei2, zrows)
  t1, u, hrs = _tc_gates(accC, v2, H, deg_col, W1, b1, W2, b2)
  accE = _make_prop_kernel(CHUNKS_E, ECHUNKS // NC, False)(
      hrs, ei2, zrows)
  return _tc_out(accE, hrs, t1, u, H, deg_col, W2)
